# Initial kernel scaffold; baseline (speedup 1.0000x reference)
import jax
import jax.numpy as jnp
from jax.experimental import pallas as pl


def kernel(x, edge_index, edge_weight, attention, Wz, bz, Wr, br, Wh, bh, Wlz, blz, Wlr, blr, Wlh, blh, fc1_w, fc1_b, fc2_w, fc2_b):
    raise NotImplementedError("write your pallas kernel here")


# speedup vs baseline: 166.1319x; 166.1319x over previous
import functools

import jax
import jax.numpy as jnp
from jax import lax
from jax.experimental import pallas as pl
from jax.experimental.pallas import tpu as pltpu
from jax.experimental.pallas import tpu_sc as plsc

N = 50000
E = 800000
F = 5
T = 12
H = 64

NPAD = 50176

DEG_TILES = 32
EPAD_DEG = 802816
DEG_EPT = EPAD_DEG // DEG_TILES
DEG_NB = DEG_EPT // 128
DEG_ROWS = NPAD // 16

EPAD_SP = 819200
SP_EPT = EPAD_SP // 16
SP_NB = SP_EPT // 128
CHB = 2
NCH = SP_NB // CHB

@functools.lru_cache(maxsize=None)
def _sc_mesh():
    return plsc.VectorSubcoreMesh(core_axis_name="c", subcore_axis_name="s",
                                  num_cores=2, num_subcores=16)


def _deg_body(dst_hbm, w_hbm, out_hbm, dstv, wv, rowv, expA, expB, zb,
              deg_sh, semA, semB):
    c = lax.axis_index("c")
    s = lax.axis_index("s")
    wid = c * 16 + s
    pltpu.sync_copy(dst_hbm.at[pl.ds(wid * DEG_EPT, DEG_EPT)], dstv)
    pltpu.sync_copy(w_hbm.at[pl.ds(wid * DEG_EPT, DEG_EPT)], wv)

    zeros16 = jnp.zeros((16,), jnp.float32)
    rows_per_tile = DEG_ROWS // 16

    @plsc.parallel_loop(0, rows_per_tile)
    def _(i):
        zb[i, :] = zeros16

    pltpu.sync_copy(zb, deg_sh.at[pl.ds(s * rows_per_tile, rows_per_tile)])
    plsc.subcore_barrier()

    def _rowv_body(b, _):
        for l in range(8):
            v = dstv[pl.ds(b * 128 + l * 16, 16)]
            rowv[b, pl.ds(l * 16, 16)] = jnp.right_shift(v, 4)
        return 0
    lax.fori_loop(0, DEG_NB, _rowv_body, 0)

    iota16 = lax.iota(jnp.int32, 16)

    def _one_batch(b, exp, sem, first):
        @pl.when(jnp.logical_not(first))
        def _():
            pltpu.make_async_copy(exp, deg_sh.at[rowv.at[b]], sem).wait()

        @plsc.parallel_loop(0, 128, unroll=2)
        def _(j):
            jj = jnp.full((16,), b * 128 + j, jnp.int32)
            dv = plsc.load_gather(dstv, [jj])
            wvv = plsc.load_gather(wv, [jj])
            row = jnp.where(iota16 == jnp.bitwise_and(dv, 15), wvv, 0.0)
            exp[j, :] = row

        pltpu.async_copy(exp, deg_sh.at[rowv.at[b]], sem, add=True)

    def _pair_body(k, _):
        _one_batch(2 * k, expA, semA, k == 0)
        _one_batch(2 * k + 1, expB, semB, k == 0)
        return 0
    lax.fori_loop(0, DEG_NB // 2, _pair_body, 0)

    pltpu.make_async_copy(expA, deg_sh.at[rowv.at[0]], semA).wait()
    pltpu.make_async_copy(expB, deg_sh.at[rowv.at[0]], semB).wait()
    plsc.subcore_barrier()
    pltpu.sync_copy(deg_sh.at[pl.ds(s * rows_per_tile, rows_per_tile)],
                    out_hbm.at[c, pl.ds(s * rows_per_tile, rows_per_tile)])


@functools.lru_cache(maxsize=None)
def _deg_kernel():
  return pl.kernel(
    _deg_body,
    out_type=jax.ShapeDtypeStruct((2, DEG_ROWS, 16), jnp.float32),
    mesh=_sc_mesh(),
    compiler_params=pltpu.CompilerParams(needs_layout_passes=False, use_tc_tiling_on_sc=False),
    scratch_types=[
        pltpu.VMEM((DEG_EPT,), jnp.int32),
        pltpu.VMEM((DEG_EPT,), jnp.float32),
        pltpu.VMEM((DEG_NB, 128), jnp.int32),
        pltpu.VMEM((128, 16), jnp.float32),
        pltpu.VMEM((128, 16), jnp.float32),
        pltpu.VMEM((DEG_ROWS // 16, 16), jnp.float32),
        pltpu.VMEM_SHARED((DEG_ROWS, 16), jnp.float32),
        pltpu.SemaphoreType.DMA,
        pltpu.SemaphoreType.DMA,
    ],
  )


def _spmm_body(y_hbm, src_hbm, dst_hbm, w_hbm, out_hbm,
               srcA, dstA, wA, srcB, dstB, wB, rowsA, rowsB,
               acc_sh, gatA, gatB, scatA, scatB, idxA, idxB):
    c = lax.axis_index("c")
    s = lax.axis_index("s")
    offc = c * NPAD
    rows_per_tile = NPAD // 16

    zeros16 = jnp.zeros((16,), jnp.float32)

    @plsc.parallel_loop(0, 128)
    def _(j):
        rowsA[0, j, pl.ds(0, 16)] = zeros16
        rowsA[0, j, pl.ds(16, 16)] = zeros16

    for m in range(rows_per_tile // 128):
        pltpu.sync_copy(rowsA.at[0],
                        acc_sh.at[pl.ds(s * rows_per_tile + m * 128, 128)])
    pltpu.sync_copy(
        rowsA.at[0, pl.ds(0, rows_per_tile % 128)],
        acc_sh.at[pl.ds(s * rows_per_tile + (rows_per_tile // 128) * 128,
                        rows_per_tile % 128)])
    plsc.subcore_barrier()

    def _issue_idx(g, srcR, dstR, wR, sem):
        return [
            pltpu.async_copy(src_hbm.at[s, pl.ds(g * CHB, CHB)], srcR, sem),
            pltpu.async_copy(dst_hbm.at[s, pl.ds(g * CHB, CHB)], dstR, sem),
            pltpu.async_copy(w_hbm.at[s, pl.ds(g * CHB, CHB)], wR, sem),
        ]

    def _wait_idx(g, srcR, dstR, wR, sem):
        pltpu.make_async_copy(src_hbm.at[s, pl.ds(g * CHB, CHB)], srcR,
                              sem).wait()
        pltpu.make_async_copy(dst_hbm.at[s, pl.ds(g * CHB, CHB)], dstR,
                              sem).wait()
        pltpu.make_async_copy(w_hbm.at[s, pl.ds(g * CHB, CHB)], wR,
                              sem).wait()

    def _offset_add(srcR):
        for a in range(CHB):
            for l in range(8):
                srcR[a, pl.ds(l * 16, 16)] = (
                    srcR[a, pl.ds(l * 16, 16)] + offc)

    def _scale_and_scatter(rowsR, wR, dstR, sem):
        descs = []
        for b in range(CHB):
            @plsc.parallel_loop(0, 128, unroll=4)
            def _(j, _b=b):
                jv = jnp.full((16,), j, jnp.int32)
                wv = plsc.load_gather(
                    wR, [jnp.full((16,), _b, jnp.int32), jv])
                rowsR[_b, j, pl.ds(0, 16)] = rowsR[_b, j, pl.ds(0, 16)] * wv
                rowsR[_b, j, pl.ds(16, 16)] = rowsR[_b, j, pl.ds(16, 16)] * wv
            descs.append(pltpu.async_copy(rowsR.at[b], acc_sh.at[dstR.at[b]],
                                          sem, add=True))
        return descs

    def _fire_gathers(srcR, rowsR, sem):
        return [pltpu.async_copy(y_hbm.at[srcR.at[b]], rowsR.at[b], sem)
                for b in range(CHB)]

    _issue_idx(0, srcA, dstA, wA, idxA)

    def _pair(k, _):
        gA = 2 * k
        gB = 2 * k + 1
        _wait_idx(gA, srcA, dstA, wA, idxA)
        _offset_add(srcA)
        gdA = _fire_gathers(srcA, rowsA, gatA)

        @pl.when(k > 0)
        def _():
            for b in range(CHB):
                pltpu.make_async_copy(rowsB.at[b], acc_sh.at[dstB.at[b]],
                                      scatB).wait()
        idB = _issue_idx(gB, srcB, dstB, wB, idxB)

        for b in range(CHB):
            gdA[b].wait()
        sdA = _scale_and_scatter(rowsA, wA, dstA, scatA)

        for d in idB:
            d.wait()
        _offset_add(srcB)
        gdB = _fire_gathers(srcB, rowsB, gatB)

        for d in sdA:
            d.wait()

        @pl.when(k < NCH // 2 - 1)
        def _():
            _issue_idx(gA + 2, srcA, dstA, wA, idxA)

        for b in range(CHB):
            gdB[b].wait()
        _scale_and_scatter(rowsB, wB, dstB, scatB)
        return 0

    lax.fori_loop(0, NCH // 2, _pair, 0)

    for b in range(CHB):
        pltpu.make_async_copy(rowsB.at[b], acc_sh.at[dstB.at[b]],
                              scatB).wait()
    plsc.subcore_barrier()
    pltpu.sync_copy(acc_sh.at[pl.ds(s * rows_per_tile, rows_per_tile)],
                    out_hbm.at[c, pl.ds(s * rows_per_tile, rows_per_tile)])


@functools.lru_cache(maxsize=None)
def _spmm_kernel():
  return pl.kernel(
    _spmm_body,
    out_type=jax.ShapeDtypeStruct((2, NPAD, 32), jnp.float32),
    mesh=_sc_mesh(),
    compiler_params=pltpu.CompilerParams(needs_layout_passes=False, use_tc_tiling_on_sc=False),
    scratch_types=[
        pltpu.VMEM((CHB, 128), jnp.int32),
        pltpu.VMEM((CHB, 128), jnp.int32),
        pltpu.VMEM((CHB, 128), jnp.float32),
        pltpu.VMEM((CHB, 128), jnp.int32),
        pltpu.VMEM((CHB, 128), jnp.int32),
        pltpu.VMEM((CHB, 128), jnp.float32),
        pltpu.VMEM((CHB, 128, 32), jnp.float32),
        pltpu.VMEM((CHB, 128, 32), jnp.float32),
        pltpu.VMEM_SHARED((NPAD, 32), jnp.float32),
        pltpu.SemaphoreType.DMA,
        pltpu.SemaphoreType.DMA,
        pltpu.SemaphoreType.DMA,
        pltpu.SemaphoreType.DMA,
        pltpu.SemaphoreType.DMA,
        pltpu.SemaphoreType.DMA,
    ],
  )


BN = 1024
_GRID = NPAD // BN


def _t1_body(d0_ref, d1_ref, x_ref, y_ref, dinv_ref):
    dc = lax.rsqrt(d0_ref[...] + d1_ref[...] + 1.0)
    dinv_ref[...] = dc
    xb = x_ref[...]
    z2 = jnp.zeros((BN, 2), jnp.float32)
    y_ref[0] = jnp.concatenate([xb[:, 0:30] * dc, z2], axis=1)
    y_ref[1] = jnp.concatenate([xb[:, 30:60] * dc, z2], axis=1)


def _make_t1(interpret=False):
  return pl.pallas_call(
    _t1_body,
    interpret=interpret,
    grid=(_GRID,),
    in_specs=[
        pl.BlockSpec((BN, 1), lambda i: (i, 0)),
        pl.BlockSpec((BN, 1), lambda i: (i, 0)),
        pl.BlockSpec((BN, 64), lambda i: (i, 0)),
    ],
    out_specs=[
        pl.BlockSpec((2, BN, 32), lambda i: (0, i, 0)),
        pl.BlockSpec((BN, 1), lambda i: (i, 0)),
    ],
    out_shape=[
        jax.ShapeDtypeStruct((2, NPAD, 32), jnp.float32),
        jax.ShapeDtypeStruct((NPAD, 1), jnp.float32),
    ],
  )


_t1_kernel = _make_t1()


def _head_body(spmm_ref, y_ref, dinv_ref, attp_ref, Wz_ref, Wh_ref,
               WlzT_ref, WlhT_ref, bz_ref, bh_ref, blz_ref, blh_ref,
               fc1w_ref, fc1b_ref, fc2r_ref, fc2b_ref, hacc_ref, o_ref):
    f32 = jnp.float32
    d = dinv_ref[...]

    att = attp_ref[...]
    m = jnp.max(att, axis=1, keepdims=True)
    ee = jnp.exp(att - m)
    pr = ee / jnp.sum(ee, axis=1, keepdims=True)

    WlzT = WlzT_ref[...]
    WlhT = WlhT_ref[...]
    Az = jnp.dot(Wz_ref[...], WlzT, preferred_element_type=f32)
    Ah = jnp.dot(Wh_ref[...], WlhT, preferred_element_type=f32)
    cz = jnp.dot(bz_ref[...], WlzT, preferred_element_type=f32) + blz_ref[...]
    ch = jnp.dot(bh_ref[...], WlhT, preferred_element_type=f32) + blh_ref[...]

    z8 = jnp.zeros((BN, 8), f32)
    a0 = jnp.concatenate([d * (spmm_ref[0] + y_ref[0]), z8], axis=1)
    a1 = jnp.concatenate([d * (spmm_ref[1] + y_ref[1]), z8], axis=1)

    hacc = jnp.zeros((BN, H), f32)
    for t in range(T):
        src = a0 if t < 6 else a1
        tau = t % 6
        at8 = src[:, 5 * tau:5 * tau + 8]
        zp = jnp.dot(at8, Az, preferred_element_type=f32) + cz
        hp = jnp.dot(at8, Ah, preferred_element_type=f32) + ch
        pt = pr[0:1, t:t + 1]
        hacc = hacc + pt * (1.0 - jax.nn.sigmoid(zp)) * jnp.tanh(hp)

    hacc_ref[...] = hacc
    hid = jax.nn.relu(
        jnp.dot(hacc, fc1w_ref[...], preferred_element_type=f32)
        + fc1b_ref[...])
    o = jnp.sum(hid * fc2r_ref[...], axis=1, keepdims=True)
    o_ref[...] = o + fc2b_ref[0:1, 0:1]


def _full(shape):
    return pl.BlockSpec(shape, lambda i: tuple(0 for _ in shape))


def _make_head(interpret=False):
  return pl.pallas_call(
    _head_body,
    interpret=interpret,
    grid=(_GRID,),
    in_specs=[
        pl.BlockSpec((2, BN, 32), lambda i: (0, i, 0)),
        pl.BlockSpec((2, BN, 32), lambda i: (0, i, 0)),
        pl.BlockSpec((BN, 1), lambda i: (i, 0)),
        _full((1, 128)),
        _full((8, 64)),
        _full((8, 64)),
        _full((64, 64)),
        _full((64, 64)),
        _full((1, 64)),
        _full((1, 64)),
        _full((1, 64)),
        _full((1, 64)),
        _full((64, 32)),
        _full((1, 32)),
        _full((1, 32)),
        _full((1, 128)),
    ],
    out_specs=[
        pl.BlockSpec((BN, 64), lambda i: (i, 0)),
        pl.BlockSpec((BN, 1), lambda i: (i, 0)),
    ],
    out_shape=[
        jax.ShapeDtypeStruct((NPAD, 64), jnp.float32),
        jax.ShapeDtypeStruct((NPAD, 1), jnp.float32),
    ],
  )


_head_kernel = _make_head()


def kernel(x, edge_index, edge_weight, attention, Wz, bz, Wr, br, Wh, bh,
           Wlz, blz, Wlr, blr, Wlh, blh, fc1_w, fc1_b, fc2_w, fc2_b):
    f32 = jnp.float32
    src = edge_index[0].astype(jnp.int32)
    dst = edge_index[1].astype(jnp.int32)
    w = edge_weight.astype(f32)

    padE = EPAD_SP - E
    srcp = jnp.pad(src, (0, padE))
    dstp = jnp.pad(dst, (0, padE))
    wp = jnp.pad(w, (0, padE))

    deg2 = _deg_kernel()(dstp[:EPAD_DEG], wp[:EPAD_DEG])
    d0 = deg2[0].reshape(NPAD, 1)
    d1 = deg2[1].reshape(NPAD, 1)

    xf = x.reshape(N, T * F).astype(f32)
    xpad = jnp.pad(xf, ((0, NPAD - N), (0, 4)))
    yv, dinv = _t1_kernel(d0, d1, xpad)

    yflat = yv.reshape(2 * NPAD, 32)
    spmm = _spmm_kernel()(yflat,
                          srcp.reshape(16, SP_NB, 128),
                          dstp.reshape(16, SP_NB, 128),
                          wp.reshape(16, SP_NB, 128))

    attp = jnp.pad(attention.astype(f32), (0, 128 - T),
                   constant_values=-1e30).reshape(1, 128)
    Wz8 = jnp.pad(Wz.astype(f32), ((0, 3), (0, 0)))
    Wh8 = jnp.pad(Wh.astype(f32), ((0, 3), (0, 0)))
    hacc_p, o_p = _head_kernel(
        spmm, yv, dinv, attp, Wz8, Wh8,
        Wlz[:H].astype(f32), Wlh[:H].astype(f32),
        bz.reshape(1, H).astype(f32), bh.reshape(1, H).astype(f32),
        blz.reshape(1, H).astype(f32), blh.reshape(1, H).astype(f32),
        fc1_w.astype(f32), fc1_b.reshape(1, 32).astype(f32),
        fc2_w.reshape(1, 32).astype(f32),
        jnp.pad(fc2_b.astype(f32), (0, 127)).reshape(1, 128))

    return o_p[:N], hacc_p[:N]


# speedup vs baseline: 175.9541x; 1.0591x over previous
import functools

import jax
import jax.numpy as jnp
from jax import lax
from jax.experimental import pallas as pl
from jax.experimental.pallas import tpu as pltpu
from jax.experimental.pallas import tpu_sc as plsc

N = 50000
E = 800000
F = 5
T = 12
H = 64

NPAD = 50176

DEG_TILES = 32
EPAD_DEG = 802816
DEG_EPT = EPAD_DEG // DEG_TILES
DEG_NB = DEG_EPT // 128
DEG_ROWS = NPAD // 16

EPAD_SP = 823296
SP_EPT = EPAD_SP // 16
SP_NB = SP_EPT // 128
CHB = 3
NCH = SP_NB // CHB

@functools.lru_cache(maxsize=None)
def _sc_mesh():
    return plsc.VectorSubcoreMesh(core_axis_name="c", subcore_axis_name="s",
                                  num_cores=2, num_subcores=16)


def _deg_body(dst_hbm, w_hbm, out_hbm, dstv, wv, rowv, expA, expB, zb,
              deg_sh, semA, semB):
    c = lax.axis_index("c")
    s = lax.axis_index("s")
    wid = c * 16 + s
    pltpu.sync_copy(dst_hbm.at[pl.ds(wid * DEG_EPT, DEG_EPT)], dstv)
    pltpu.sync_copy(w_hbm.at[pl.ds(wid * DEG_EPT, DEG_EPT)], wv)

    zeros16 = jnp.zeros((16,), jnp.float32)
    rows_per_tile = DEG_ROWS // 16

    @plsc.parallel_loop(0, rows_per_tile)
    def _(i):
        zb[i, :] = zeros16

    pltpu.sync_copy(zb, deg_sh.at[pl.ds(s * rows_per_tile, rows_per_tile)])
    plsc.subcore_barrier()

    def _rowv_body(b, _):
        for l in range(8):
            v = dstv[pl.ds(b * 128 + l * 16, 16)]
            rowv[b, pl.ds(l * 16, 16)] = jnp.right_shift(v, 4)
        return 0
    lax.fori_loop(0, DEG_NB, _rowv_body, 0)

    iota16 = lax.iota(jnp.int32, 16)

    def _one_batch(b, exp, sem, first):
        @pl.when(jnp.logical_not(first))
        def _():
            pltpu.make_async_copy(exp, deg_sh.at[rowv.at[b]], sem).wait()

        @plsc.parallel_loop(0, 128, unroll=2)
        def _(j):
            jj = jnp.full((16,), b * 128 + j, jnp.int32)
            dv = plsc.load_gather(dstv, [jj])
            wvv = plsc.load_gather(wv, [jj])
            row = jnp.where(iota16 == jnp.bitwise_and(dv, 15), wvv, 0.0)
            exp[j, :] = row

        pltpu.async_copy(exp, deg_sh.at[rowv.at[b]], sem, add=True)

    def _pair_body(k, _):
        _one_batch(2 * k, expA, semA, k == 0)
        _one_batch(2 * k + 1, expB, semB, k == 0)
        return 0
    lax.fori_loop(0, DEG_NB // 2, _pair_body, 0)

    pltpu.make_async_copy(expA, deg_sh.at[rowv.at[0]], semA).wait()
    pltpu.make_async_copy(expB, deg_sh.at[rowv.at[0]], semB).wait()
    plsc.subcore_barrier()
    pltpu.sync_copy(deg_sh.at[pl.ds(s * rows_per_tile, rows_per_tile)],
                    out_hbm.at[c, pl.ds(s * rows_per_tile, rows_per_tile)])


@functools.lru_cache(maxsize=None)
def _deg_kernel():
  return pl.kernel(
    _deg_body,
    out_type=jax.ShapeDtypeStruct((2, DEG_ROWS, 16), jnp.float32),
    mesh=_sc_mesh(),
    compiler_params=pltpu.CompilerParams(needs_layout_passes=False, use_tc_tiling_on_sc=False),
    scratch_types=[
        pltpu.VMEM((DEG_EPT,), jnp.int32),
        pltpu.VMEM((DEG_EPT,), jnp.float32),
        pltpu.VMEM((DEG_NB, 128), jnp.int32),
        pltpu.VMEM((128, 16), jnp.float32),
        pltpu.VMEM((128, 16), jnp.float32),
        pltpu.VMEM((DEG_ROWS // 16, 16), jnp.float32),
        pltpu.VMEM_SHARED((DEG_ROWS, 16), jnp.float32),
        pltpu.SemaphoreType.DMA,
        pltpu.SemaphoreType.DMA,
    ],
  )


def _spmm_body(y_hbm, src_hbm, src2_hbm, dst_hbm, w_hbm, out_hbm,
               srcA, dstA, wA, srcB, dstB, wB, rowsA, rowsB,
               acc_sh, gatA, gatB, scatA, scatB, idxA, idxB):
    c = lax.axis_index("c")
    s = lax.axis_index("s")
    rows_per_tile = NPAD // 16

    zeros16 = jnp.zeros((16,), jnp.float32)

    @plsc.parallel_loop(0, 128)
    def _(j):
        rowsA[0, j, pl.ds(0, 16)] = zeros16
        rowsA[0, j, pl.ds(16, 16)] = zeros16

    for m in range(rows_per_tile // 128):
        pltpu.sync_copy(rowsA.at[0],
                        acc_sh.at[pl.ds(s * rows_per_tile + m * 128, 128)])
    pltpu.sync_copy(
        rowsA.at[0, pl.ds(0, rows_per_tile % 128)],
        acc_sh.at[pl.ds(s * rows_per_tile + (rows_per_tile // 128) * 128,
                        rows_per_tile % 128)])
    plsc.subcore_barrier()

    def _issue_idx(g, srcR, dstR, wR, sem):
        @pl.when(c == 0)
        def _():
            pltpu.async_copy(src_hbm.at[s, pl.ds(g * CHB, CHB)], srcR, sem)

        @pl.when(c == 1)
        def _():
            pltpu.async_copy(src2_hbm.at[s, pl.ds(g * CHB, CHB)], srcR, sem)
        pltpu.async_copy(dst_hbm.at[s, pl.ds(g * CHB, CHB)], dstR, sem)
        pltpu.async_copy(w_hbm.at[s, pl.ds(g * CHB, CHB)], wR, sem)

    def _wait_idx(g, srcR, dstR, wR, sem):
        pltpu.make_async_copy(src_hbm.at[s, pl.ds(g * CHB, CHB)], srcR,
                              sem).wait()
        pltpu.make_async_copy(dst_hbm.at[s, pl.ds(g * CHB, CHB)], dstR,
                              sem).wait()
        pltpu.make_async_copy(w_hbm.at[s, pl.ds(g * CHB, CHB)], wR,
                              sem).wait()

    def _scale_and_scatter(gdescs, rowsR, wR, dstR, sem):
        descs = []
        for b in range(CHB):
            gdescs[b].wait()

            @plsc.parallel_loop(0, 128, unroll=8)
            def _(j, _b=b):
                jv = jnp.full((16,), j, jnp.int32)
                wv = plsc.load_gather(
                    wR, [jnp.full((16,), _b, jnp.int32), jv])
                rowsR[_b, j, pl.ds(0, 16)] = rowsR[_b, j, pl.ds(0, 16)] * wv
                rowsR[_b, j, pl.ds(16, 16)] = rowsR[_b, j, pl.ds(16, 16)] * wv
            descs.append(pltpu.async_copy(rowsR.at[b], acc_sh.at[dstR.at[b]],
                                          sem, add=True))
        return descs

    def _fire_gathers(srcR, rowsR, sem):
        return [pltpu.async_copy(y_hbm.at[srcR.at[b]], rowsR.at[b], sem)
                for b in range(CHB)]

    _issue_idx(0, srcA, dstA, wA, idxA)

    def _pair(k, _):
        gA = 2 * k
        gB = 2 * k + 1
        _wait_idx(gA, srcA, dstA, wA, idxA)
        gdA = _fire_gathers(srcA, rowsA, gatA)

        @pl.when(k > 0)
        def _():
            for b in range(CHB):
                pltpu.make_async_copy(rowsB.at[b], acc_sh.at[dstB.at[b]],
                                      scatB).wait()
        _issue_idx(gB, srcB, dstB, wB, idxB)

        sdA = _scale_and_scatter(gdA, rowsA, wA, dstA, scatA)

        _wait_idx(gB, srcB, dstB, wB, idxB)
        gdB = _fire_gathers(srcB, rowsB, gatB)

        for d in sdA:
            d.wait()

        @pl.when(k < NCH // 2 - 1)
        def _():
            _issue_idx(gA + 2, srcA, dstA, wA, idxA)

        _scale_and_scatter(gdB, rowsB, wB, dstB, scatB)
        return 0

    lax.fori_loop(0, NCH // 2, _pair, 0)

    for b in range(CHB):
        pltpu.make_async_copy(rowsB.at[b], acc_sh.at[dstB.at[b]],
                              scatB).wait()
    plsc.subcore_barrier()
    pltpu.sync_copy(acc_sh.at[pl.ds(s * rows_per_tile, rows_per_tile)],
                    out_hbm.at[c, pl.ds(s * rows_per_tile, rows_per_tile)])


@functools.lru_cache(maxsize=None)
def _spmm_kernel():
  return pl.kernel(
    _spmm_body,
    out_type=jax.ShapeDtypeStruct((2, NPAD, 32), jnp.float32),
    mesh=_sc_mesh(),
    compiler_params=pltpu.CompilerParams(needs_layout_passes=False, use_tc_tiling_on_sc=False),
    scratch_types=[
        pltpu.VMEM((CHB, 128), jnp.int32),
        pltpu.VMEM((CHB, 128), jnp.int32),
        pltpu.VMEM((CHB, 128), jnp.float32),
        pltpu.VMEM((CHB, 128), jnp.int32),
        pltpu.VMEM((CHB, 128), jnp.int32),
        pltpu.VMEM((CHB, 128), jnp.float32),
        pltpu.VMEM((CHB, 128, 32), jnp.float32),
        pltpu.VMEM((CHB, 128, 32), jnp.float32),
        pltpu.VMEM_SHARED((NPAD, 32), jnp.float32),
        pltpu.SemaphoreType.DMA,
        pltpu.SemaphoreType.DMA,
        pltpu.SemaphoreType.DMA,
        pltpu.SemaphoreType.DMA,
        pltpu.SemaphoreType.DMA,
        pltpu.SemaphoreType.DMA,
    ],
  )


BN = 1024
_GRID = NPAD // BN
BN2 = 1024
_GRID2 = NPAD // BN2


def _t1_body(d0_ref, d1_ref, x_ref, y_ref, dinv_ref):
    dc = lax.rsqrt(d0_ref[...] + d1_ref[...] + 1.0)
    dinv_ref[...] = dc
    xb = x_ref[...]
    z2 = jnp.zeros((BN, 2), jnp.float32)
    y_ref[0] = jnp.concatenate([xb[:, 0:30] * dc, z2], axis=1)
    y_ref[1] = jnp.concatenate([xb[:, 30:60] * dc, z2], axis=1)


def _make_t1(interpret=False):
  return pl.pallas_call(
    _t1_body,
    interpret=interpret,
    grid=(_GRID,),
    in_specs=[
        pl.BlockSpec((BN, 1), lambda i: (i, 0)),
        pl.BlockSpec((BN, 1), lambda i: (i, 0)),
        pl.BlockSpec((BN, 64), lambda i: (i, 0)),
    ],
    out_specs=[
        pl.BlockSpec((2, BN, 32), lambda i: (0, i, 0)),
        pl.BlockSpec((BN, 1), lambda i: (i, 0)),
    ],
    out_shape=[
        jax.ShapeDtypeStruct((2, NPAD, 32), jnp.float32),
        jax.ShapeDtypeStruct((NPAD, 1), jnp.float32),
    ],
  )


_t1_kernel = _make_t1()


def _head_body(spmm_ref, y_ref, dinv_ref, attp_ref, Wz_ref, Wh_ref,
               WlzT_ref, WlhT_ref, bz_ref, bh_ref, blz_ref, blh_ref,
               fc1w_ref, fc1b_ref, fc2r_ref, fc2b_ref, hacc_ref, o_ref):
    f32 = jnp.float32
    d = dinv_ref[...]

    att = attp_ref[...]
    m = jnp.max(att, axis=1, keepdims=True)
    ee = jnp.exp(att - m)
    pr = ee / jnp.sum(ee, axis=1, keepdims=True)

    WlzT = WlzT_ref[...]
    WlhT = WlhT_ref[...]
    Az = jnp.dot(Wz_ref[...], WlzT, preferred_element_type=f32)
    Ah = jnp.dot(Wh_ref[...], WlhT, preferred_element_type=f32)
    cz = jnp.dot(bz_ref[...], WlzT, preferred_element_type=f32) + blz_ref[...]
    ch = jnp.dot(bh_ref[...], WlhT, preferred_element_type=f32) + blh_ref[...]

    z8 = jnp.zeros((BN2, 8), f32)
    a0 = jnp.concatenate([d * (spmm_ref[0] + y_ref[0]), z8], axis=1)
    a1 = jnp.concatenate([d * (spmm_ref[1] + y_ref[1]), z8], axis=1)

    hacc = jnp.zeros((BN2, H), f32)
    for t in range(T):
        src = a0 if t < 6 else a1
        tau = t % 6
        at8 = src[:, 5 * tau:5 * tau + 8]
        zp = jnp.dot(at8, Az, preferred_element_type=f32) + cz
        hp = jnp.dot(at8, Ah, preferred_element_type=f32) + ch
        pt = pr[0:1, t:t + 1]
        gate = 1.0 + jnp.tanh(-0.5 * zp)
        hacc = hacc + (0.5 * pt) * gate * jnp.tanh(hp)

    hacc_ref[...] = hacc
    hid = jax.nn.relu(
        jnp.dot(hacc, fc1w_ref[...], preferred_element_type=f32)
        + fc1b_ref[...])
    o = jnp.sum(hid * fc2r_ref[...], axis=1, keepdims=True)
    o_ref[...] = o + fc2b_ref[0:1, 0:1]


def _full(shape):
    return pl.BlockSpec(shape, lambda i: tuple(0 for _ in shape))


def _make_head(interpret=False):
  return pl.pallas_call(
    _head_body,
    interpret=interpret,
    grid=(_GRID2,),
    in_specs=[
        pl.BlockSpec((2, BN2, 32), lambda i: (0, i, 0)),
        pl.BlockSpec((2, BN2, 32), lambda i: (0, i, 0)),
        pl.BlockSpec((BN2, 1), lambda i: (i, 0)),
        _full((1, 128)),
        _full((8, 64)),
        _full((8, 64)),
        _full((64, 64)),
        _full((64, 64)),
        _full((1, 64)),
        _full((1, 64)),
        _full((1, 64)),
        _full((1, 64)),
        _full((64, 32)),
        _full((1, 32)),
        _full((1, 32)),
        _full((1, 128)),
    ],
    out_specs=[
        pl.BlockSpec((BN2, 64), lambda i: (i, 0)),
        pl.BlockSpec((BN2, 1), lambda i: (i, 0)),
    ],
    out_shape=[
        jax.ShapeDtypeStruct((NPAD, 64), jnp.float32),
        jax.ShapeDtypeStruct((NPAD, 1), jnp.float32),
    ],
  )


_head_kernel = _make_head()


def kernel(x, edge_index, edge_weight, attention, Wz, bz, Wr, br, Wh, bh,
           Wlz, blz, Wlr, blr, Wlh, blh, fc1_w, fc1_b, fc2_w, fc2_b):
    f32 = jnp.float32
    src = edge_index[0].astype(jnp.int32)
    dst = edge_index[1].astype(jnp.int32)
    w = edge_weight.astype(f32)

    padE = EPAD_SP - E
    srcp = jnp.pad(src, (0, padE))
    dstp = jnp.pad(dst, (0, padE))
    wp = jnp.pad(w, (0, padE))

    deg2 = _deg_kernel()(dstp[:EPAD_DEG], wp[:EPAD_DEG])
    d0 = deg2[0].reshape(NPAD, 1)
    d1 = deg2[1].reshape(NPAD, 1)

    xf = x.reshape(N, T * F).astype(f32)
    xpad = jnp.pad(xf, ((0, NPAD - N), (0, 4)))
    yv, dinv = _t1_kernel(d0, d1, xpad)

    yflat = yv.reshape(2 * NPAD, 32)
    spmm = _spmm_kernel()(yflat,
                          srcp.reshape(16, SP_NB, 128),
                          (srcp + NPAD).reshape(16, SP_NB, 128),
                          dstp.reshape(16, SP_NB, 128),
                          wp.reshape(16, SP_NB, 128))

    attp = jnp.pad(attention.astype(f32), (0, 128 - T),
                   constant_values=-1e30).reshape(1, 128)
    Wz8 = jnp.pad(Wz.astype(f32), ((0, 3), (0, 0)))
    Wh8 = jnp.pad(Wh.astype(f32), ((0, 3), (0, 0)))
    hacc_p, o_p = _head_kernel(
        spmm, yv, dinv, attp, Wz8, Wh8,
        Wlz[:H].astype(f32), Wlh[:H].astype(f32),
        bz.reshape(1, H).astype(f32), bh.reshape(1, H).astype(f32),
        blz.reshape(1, H).astype(f32), blh.reshape(1, H).astype(f32),
        fc1_w.astype(f32), fc1_b.reshape(1, 32).astype(f32),
        fc2_w.reshape(1, 32).astype(f32),
        jnp.pad(fc2_b.astype(f32), (0, 127)).reshape(1, 128))

    return o_p[:N], hacc_p[:N]


# speedup vs baseline: 187.6957x; 1.0667x over previous
import functools

import jax
import jax.numpy as jnp
from jax import lax
from jax.experimental import pallas as pl
from jax.experimental.pallas import tpu as pltpu
from jax.experimental.pallas import tpu_sc as plsc

N = 50000
E = 800000
F = 5
T = 12
H = 64

NPAD = 50176

DEG_TILES = 32
EPAD_DEG = 802816
DEG_EPT = EPAD_DEG // DEG_TILES
DEG_NB = DEG_EPT // 128
DEG_ROWS = NPAD // 16

EPAD_SP = 823296
SP_EPT = EPAD_SP // 16
SP_NB = SP_EPT // 128
CHB = 3
NCH = SP_NB // CHB

@functools.lru_cache(maxsize=None)
def _sc_mesh():
    return plsc.VectorSubcoreMesh(core_axis_name="c", subcore_axis_name="s",
                                  num_cores=2, num_subcores=16)


def _deg_body(dst_hbm, w_hbm, out_hbm, dstv, wv, rowv, expA, expB, zb,
              deg_sh, semA, semB):
    c = lax.axis_index("c")
    s = lax.axis_index("s")
    wid = c * 16 + s
    pltpu.sync_copy(dst_hbm.at[pl.ds(wid * DEG_EPT, DEG_EPT)], dstv)
    pltpu.sync_copy(w_hbm.at[pl.ds(wid * DEG_EPT, DEG_EPT)], wv)

    zeros16 = jnp.zeros((16,), jnp.float32)
    rows_per_tile = DEG_ROWS // 16

    @plsc.parallel_loop(0, rows_per_tile)
    def _(i):
        zb[i, :] = zeros16

    pltpu.sync_copy(zb, deg_sh.at[pl.ds(s * rows_per_tile, rows_per_tile)])
    plsc.subcore_barrier()

    def _rowv_body(b, _):
        for l in range(8):
            v = dstv[pl.ds(b * 128 + l * 16, 16)]
            rowv[b, pl.ds(l * 16, 16)] = jnp.right_shift(v, 4)
        return 0
    lax.fori_loop(0, DEG_NB, _rowv_body, 0)

    iota16 = lax.iota(jnp.int32, 16)

    def _one_batch(b, exp, sem, first):
        @pl.when(jnp.logical_not(first))
        def _():
            pltpu.make_async_copy(exp, deg_sh.at[rowv.at[b]], sem).wait()

        @plsc.parallel_loop(0, 128, unroll=2)
        def _(j):
            jj = jnp.full((16,), b * 128 + j, jnp.int32)
            dv = plsc.load_gather(dstv, [jj])
            wvv = plsc.load_gather(wv, [jj])
            row = jnp.where(iota16 == jnp.bitwise_and(dv, 15), wvv, 0.0)
            exp[j, :] = row

        pltpu.async_copy(exp, deg_sh.at[rowv.at[b]], sem, add=True)

    def _pair_body(k, _):
        _one_batch(2 * k, expA, semA, k == 0)
        _one_batch(2 * k + 1, expB, semB, k == 0)
        return 0
    lax.fori_loop(0, DEG_NB // 2, _pair_body, 0)

    pltpu.make_async_copy(expA, deg_sh.at[rowv.at[0]], semA).wait()
    pltpu.make_async_copy(expB, deg_sh.at[rowv.at[0]], semB).wait()
    plsc.subcore_barrier()
    pltpu.sync_copy(deg_sh.at[pl.ds(s * rows_per_tile, rows_per_tile)],
                    out_hbm.at[c, pl.ds(s * rows_per_tile, rows_per_tile)])


@functools.lru_cache(maxsize=None)
def _deg_kernel():
  return pl.kernel(
    _deg_body,
    out_type=jax.ShapeDtypeStruct((2, DEG_ROWS, 16), jnp.float32),
    mesh=_sc_mesh(),
    compiler_params=pltpu.CompilerParams(needs_layout_passes=False, use_tc_tiling_on_sc=False),
    scratch_types=[
        pltpu.VMEM((DEG_EPT,), jnp.int32),
        pltpu.VMEM((DEG_EPT,), jnp.float32),
        pltpu.VMEM((DEG_NB, 128), jnp.int32),
        pltpu.VMEM((128, 16), jnp.float32),
        pltpu.VMEM((128, 16), jnp.float32),
        pltpu.VMEM((DEG_ROWS // 16, 16), jnp.float32),
        pltpu.VMEM_SHARED((DEG_ROWS, 16), jnp.float32),
        pltpu.SemaphoreType.DMA,
        pltpu.SemaphoreType.DMA,
    ],
  )


def _spmm_body(y_hbm, src_hbm, src2_hbm, dst_hbm, w_hbm, out_hbm,
               srcA, dstA, wA, srcB, dstB, wB, rowsA, rowsB,
               acc_sh, gatA, gatB, scatA, scatB, idxA, idxB):
    c = lax.axis_index("c")
    s = lax.axis_index("s")
    rows_per_tile = NPAD // 16

    zeros16 = jnp.zeros((16,), jnp.float32)

    @plsc.parallel_loop(0, 128)
    def _(j):
        rowsA[0, j, pl.ds(0, 16)] = zeros16
        rowsA[0, j, pl.ds(16, 16)] = zeros16

    for m in range(rows_per_tile // 128):
        pltpu.sync_copy(rowsA.at[0],
                        acc_sh.at[pl.ds(s * rows_per_tile + m * 128, 128)])
    pltpu.sync_copy(
        rowsA.at[0, pl.ds(0, rows_per_tile % 128)],
        acc_sh.at[pl.ds(s * rows_per_tile + (rows_per_tile // 128) * 128,
                        rows_per_tile % 128)])
    plsc.subcore_barrier()

    def _issue_idx(g, srcR, dstR, wR, sem):
        @pl.when(c == 0)
        def _():
            pltpu.async_copy(src_hbm.at[s, pl.ds(g * CHB, CHB)], srcR, sem)

        @pl.when(c == 1)
        def _():
            pltpu.async_copy(src2_hbm.at[s, pl.ds(g * CHB, CHB)], srcR, sem)
        pltpu.async_copy(dst_hbm.at[s, pl.ds(g * CHB, CHB)], dstR, sem)
        pltpu.async_copy(w_hbm.at[s, pl.ds(g * CHB, CHB)], wR, sem)

    def _wait_idx(g, srcR, dstR, wR, sem):
        pltpu.make_async_copy(src_hbm.at[s, pl.ds(g * CHB, CHB)], srcR,
                              sem).wait()
        pltpu.make_async_copy(dst_hbm.at[s, pl.ds(g * CHB, CHB)], dstR,
                              sem).wait()
        pltpu.make_async_copy(w_hbm.at[s, pl.ds(g * CHB, CHB)], wR,
                              sem).wait()

    def _scale_and_scatter(gdescs, rowsR, wR, dstR, sem):
        descs = []
        for b in range(CHB):
            gdescs[b].wait()

            @plsc.parallel_loop(0, 128, unroll=8)
            def _(j, _b=b):
                jv = jnp.full((16,), j, jnp.int32)
                wv = plsc.load_gather(
                    wR, [jnp.full((16,), _b, jnp.int32), jv])
                rowsR[_b, j, pl.ds(0, 16)] = rowsR[_b, j, pl.ds(0, 16)] * wv
                rowsR[_b, j, pl.ds(16, 16)] = rowsR[_b, j, pl.ds(16, 16)] * wv
            descs.append(pltpu.async_copy(rowsR.at[b], acc_sh.at[dstR.at[b]],
                                          sem, add=True))
        return descs

    def _fire_gathers(srcR, rowsR, sem):
        return [pltpu.async_copy(y_hbm.at[srcR.at[b]], rowsR.at[b], sem)
                for b in range(CHB)]

    _issue_idx(0, srcA, dstA, wA, idxA)

    def _pair(k, _):
        gA = 2 * k
        gB = 2 * k + 1
        _wait_idx(gA, srcA, dstA, wA, idxA)
        gdA = _fire_gathers(srcA, rowsA, gatA)

        @pl.when(k > 0)
        def _():
            for b in range(CHB):
                pltpu.make_async_copy(rowsB.at[b], acc_sh.at[dstB.at[b]],
                                      scatB).wait()
        _issue_idx(gB, srcB, dstB, wB, idxB)

        sdA = _scale_and_scatter(gdA, rowsA, wA, dstA, scatA)

        _wait_idx(gB, srcB, dstB, wB, idxB)
        gdB = _fire_gathers(srcB, rowsB, gatB)

        for d in sdA:
            d.wait()

        @pl.when(k < NCH // 2 - 1)
        def _():
            _issue_idx(gA + 2, srcA, dstA, wA, idxA)

        _scale_and_scatter(gdB, rowsB, wB, dstB, scatB)
        return 0

    lax.fori_loop(0, NCH // 2, _pair, 0)

    for b in range(CHB):
        pltpu.make_async_copy(rowsB.at[b], acc_sh.at[dstB.at[b]],
                              scatB).wait()
    plsc.subcore_barrier()
    pltpu.sync_copy(acc_sh.at[pl.ds(s * rows_per_tile, rows_per_tile)],
                    out_hbm.at[c, pl.ds(s * rows_per_tile, rows_per_tile)])


@functools.lru_cache(maxsize=None)
def _spmm_kernel():
  return pl.kernel(
    _spmm_body,
    out_type=jax.ShapeDtypeStruct((2, NPAD, 32), jnp.float32),
    mesh=_sc_mesh(),
    compiler_params=pltpu.CompilerParams(needs_layout_passes=False, use_tc_tiling_on_sc=False),
    scratch_types=[
        pltpu.VMEM((CHB, 128), jnp.int32),
        pltpu.VMEM((CHB, 128), jnp.int32),
        pltpu.VMEM((CHB, 128), jnp.float32),
        pltpu.VMEM((CHB, 128), jnp.int32),
        pltpu.VMEM((CHB, 128), jnp.int32),
        pltpu.VMEM((CHB, 128), jnp.float32),
        pltpu.VMEM((CHB, 128, 32), jnp.float32),
        pltpu.VMEM((CHB, 128, 32), jnp.float32),
        pltpu.VMEM_SHARED((NPAD, 32), jnp.float32),
        pltpu.SemaphoreType.DMA,
        pltpu.SemaphoreType.DMA,
        pltpu.SemaphoreType.DMA,
        pltpu.SemaphoreType.DMA,
        pltpu.SemaphoreType.DMA,
        pltpu.SemaphoreType.DMA,
    ],
  )


BN = 1024
_GRID = NPAD // BN
BN2 = 1024
_GRID2 = NPAD // BN2


def _t1_body(d0_ref, d1_ref, x_ref, y_ref, dinv_ref):
    dc = lax.rsqrt(d0_ref[...] + d1_ref[...] + 1.0)
    dinv_ref[...] = dc
    xb = x_ref[...]
    z2 = jnp.zeros((BN, 2), jnp.float32)
    y_ref[0] = jnp.concatenate([xb[:, 0:30] * dc, z2], axis=1)
    y_ref[1] = jnp.concatenate([xb[:, 30:60] * dc, z2], axis=1)


def _make_t1(interpret=False):
  return pl.pallas_call(
    _t1_body,
    interpret=interpret,
    grid=(_GRID,),
    in_specs=[
        pl.BlockSpec((BN, 1), lambda i: (i, 0)),
        pl.BlockSpec((BN, 1), lambda i: (i, 0)),
        pl.BlockSpec((BN, 60), lambda i: (i, 0)),
    ],
    out_specs=[
        pl.BlockSpec((2, BN, 32), lambda i: (0, i, 0)),
        pl.BlockSpec((BN, 1), lambda i: (i, 0)),
    ],
    out_shape=[
        jax.ShapeDtypeStruct((2, NPAD, 32), jnp.float32),
        jax.ShapeDtypeStruct((NPAD, 1), jnp.float32),
    ],
  )


_t1_kernel = _make_t1()


def _head_body(spmm_ref, y_ref, dinv_ref, attp_ref, Wz_ref, Wh_ref,
               WlzT_ref, WlhT_ref, bz_ref, bh_ref, blz_ref, blh_ref,
               fc1w_ref, fc1b_ref, fc2r_ref, fc2b_ref, hacc_ref, o_ref):
    f32 = jnp.float32
    d = dinv_ref[...]

    att = attp_ref[...]
    m = jnp.max(att, axis=1, keepdims=True)
    ee = jnp.exp(att - m)
    pr = ee / jnp.sum(ee, axis=1, keepdims=True)

    WlzT = WlzT_ref[...]
    WlhT = WlhT_ref[...]
    Az = jnp.dot(Wz_ref[...], WlzT, preferred_element_type=f32)
    Ah = jnp.dot(Wh_ref[...], WlhT, preferred_element_type=f32)
    cz = jnp.dot(bz_ref[...], WlzT, preferred_element_type=f32) + blz_ref[...]
    ch = jnp.dot(bh_ref[...], WlhT, preferred_element_type=f32) + blh_ref[...]

    z8 = jnp.zeros((BN2, 8), f32)
    a0 = jnp.concatenate([d * (spmm_ref[0] + y_ref[0]), z8], axis=1)
    a1 = jnp.concatenate([d * (spmm_ref[1] + y_ref[1]), z8], axis=1)

    hacc = jnp.zeros((BN2, H), f32)
    for t in range(T):
        src = a0 if t < 6 else a1
        tau = t % 6
        at8 = src[:, 5 * tau:5 * tau + 8]
        zp = jnp.dot(at8, Az, preferred_element_type=f32) + cz
        hp = jnp.dot(at8, Ah, preferred_element_type=f32) + ch
        pt = pr[0:1, t:t + 1]
        gate = 1.0 + jnp.tanh(-0.5 * zp)
        hacc = hacc + (0.5 * pt) * gate * jnp.tanh(hp)

    hacc_ref[...] = hacc
    hid = jax.nn.relu(
        jnp.dot(hacc, fc1w_ref[...], preferred_element_type=f32)
        + fc1b_ref[...])
    o = jnp.sum(hid * fc2r_ref[...], axis=1, keepdims=True)
    o_ref[...] = o + fc2b_ref[0:1, 0:1]


def _full(shape):
    return pl.BlockSpec(shape, lambda i: tuple(0 for _ in shape))


def _make_head(interpret=False):
  return pl.pallas_call(
    _head_body,
    interpret=interpret,
    grid=(_GRID2,),
    in_specs=[
        pl.BlockSpec((2, BN2, 32), lambda i: (0, i, 0)),
        pl.BlockSpec((2, BN2, 32), lambda i: (0, i, 0)),
        pl.BlockSpec((BN2, 1), lambda i: (i, 0)),
        _full((1, 128)),
        _full((8, 64)),
        _full((8, 64)),
        _full((64, 64)),
        _full((64, 64)),
        _full((1, 64)),
        _full((1, 64)),
        _full((1, 64)),
        _full((1, 64)),
        _full((64, 32)),
        _full((1, 32)),
        _full((1, 32)),
        _full((1, 128)),
    ],
    out_specs=[
        pl.BlockSpec((BN2, 64), lambda i: (i, 0)),
        pl.BlockSpec((BN2, 1), lambda i: (i, 0)),
    ],
    out_shape=[
        jax.ShapeDtypeStruct((N, 64), jnp.float32),
        jax.ShapeDtypeStruct((N, 1), jnp.float32),
    ],
  )


_head_kernel = _make_head()


def kernel(x, edge_index, edge_weight, attention, Wz, bz, Wr, br, Wh, bh,
           Wlz, blz, Wlr, blr, Wlh, blh, fc1_w, fc1_b, fc2_w, fc2_b):
    f32 = jnp.float32
    src = edge_index[0].astype(jnp.int32)
    dst = edge_index[1].astype(jnp.int32)
    w = edge_weight.astype(f32)

    padE = EPAD_SP - E
    srcp = jnp.pad(src, (0, padE))
    dstp = jnp.pad(dst, (0, padE))
    wp = jnp.pad(w, (0, padE))

    deg2 = _deg_kernel()(dstp[:EPAD_DEG], wp[:EPAD_DEG])
    d0 = deg2[0].reshape(NPAD, 1)
    d1 = deg2[1].reshape(NPAD, 1)

    xf = x.reshape(N, T * F).astype(f32)
    yv, dinv = _t1_kernel(d0, d1, xf)

    yflat = yv.reshape(2 * NPAD, 32)
    spmm = _spmm_kernel()(yflat,
                          srcp.reshape(16, SP_NB, 128),
                          (srcp + NPAD).reshape(16, SP_NB, 128),
                          dstp.reshape(16, SP_NB, 128),
                          wp.reshape(16, SP_NB, 128))

    attp = jnp.pad(attention.astype(f32), (0, 128 - T),
                   constant_values=-1e30).reshape(1, 128)
    Wz8 = jnp.pad(Wz.astype(f32), ((0, 3), (0, 0)))
    Wh8 = jnp.pad(Wh.astype(f32), ((0, 3), (0, 0)))
    hacc_p, o_p = _head_kernel(
        spmm, yv, dinv, attp, Wz8, Wh8,
        Wlz[:H].astype(f32), Wlh[:H].astype(f32),
        bz.reshape(1, H).astype(f32), bh.reshape(1, H).astype(f32),
        blz.reshape(1, H).astype(f32), blh.reshape(1, H).astype(f32),
        fc1_w.astype(f32), fc1_b.reshape(1, 32).astype(f32),
        fc2_w.reshape(1, 32).astype(f32),
        jnp.pad(fc2_b.astype(f32), (0, 127)).reshape(1, 128))

    return o_p, hacc_p
